# baseline (speedup 1.0000x reference)
import jax
import jax.numpy as jnp
from jax import lax
from jax.experimental import pallas as pl
from jax.experimental.pallas import tpu as pltpu
from jax.experimental.pallas import tpu_sc as plsc

EPS = 0.1
N = 3072
N_ROWS = N + 1
NW = 32
ROWS_PER_W = N // NW
CHUNK = 8
N_CHUNKS = ROWS_PER_W // CHUNK
L = 16
NG = N // L
STAGE = ROWS_PER_W + L


def _prep_kernel(x_ref, center_ref, err_ref, rows_ref):
    xv = x_ref[...]
    lo = jnp.maximum(EPS - xv, 0.0) * 0.5
    hi = jnp.maximum(xv - (1.0 - EPS), 0.0) * 0.5
    err = EPS - lo - hi
    cond = err >= 0.0
    ci = cond.astype(jnp.int32)
    lane = jax.lax.broadcasted_iota(jnp.int32, (1, N), 1)
    incl = ci
    s = 1
    while s < N:
        shifted = pltpu.roll(incl, s, axis=1)
        incl = incl + jnp.where(lane >= s, shifted, 0)
        s *= 2
    center_ref[...] = xv + lo - hi
    err_ref[...] = err
    rows_ref[...] = jnp.where(cond, incl, N_ROWS)


def _sc_body(x_hbm, out_hbm,
             xbuf, cbuf, rowcol, segs, zbuf,
             semi, semb, sems):
    wid = lax.axis_index("s") * 2 + lax.axis_index("c")
    r0 = wid * ROWS_PER_W
    n_owned = ROWS_PER_W + (wid + 1) // NW

    cp_x = pltpu.async_copy(x_hbm, xbuf, semi)

    lane = lax.iota(jnp.int32, L)
    z16 = jnp.zeros((L,), jnp.float32)
    neg16 = jnp.full((L,), -1, jnp.int32)

    def zero_row(i, _):
        def zero_cols(g, __):
            base = g * 4 * L
            zbuf[i, pl.ds(base, L)] = z16
            zbuf[i, pl.ds(base + L, L)] = z16
            zbuf[i, pl.ds(base + 2 * L, L)] = z16
            zbuf[i, pl.ds(base + 3 * L, L)] = z16
            return __

        return lax.fori_loop(0, NG // 4, zero_cols, _)

    lax.fori_loop(0, CHUNK, zero_row, 0)

    big = [
        pltpu.async_copy(zbuf, out_hbm.at[pl.ds(r0 + k * CHUNK, CHUNK)],
                         semb.at[k])
        for k in range(N_CHUNKS)
    ]

    def zero_stage(j, _):
        base = j * 4 * L
        segs[pl.ds(base, L)] = z16
        segs[pl.ds(base + L, L)] = z16
        segs[pl.ds(base + 2 * L, L)] = z16
        segs[pl.ds(base + 3 * L, L)] = z16
        return _

    lax.fori_loop(0, STAGE // 4, zero_stage, 0)

    def zero_rc(j, _):
        rowcol[pl.ds(j * L, L)] = neg16
        return _

    lax.fori_loop(0, STAGE // L, zero_rc, 0)

    cp_x.wait()

    def col_group(g, carry):
        xv = xbuf[pl.ds(g * L, L)]
        lo = jnp.maximum(EPS - xv, 0.0) * 0.5
        hi = jnp.maximum(xv - (1.0 - EPS), 0.0) * 0.5
        err = EPS - lo - hi
        cond = err >= 0.0
        ci = jnp.where(cond, 1, 0)
        incl = plsc.cumsum(ci)
        rows = jnp.where(cond, carry + incl, N_ROWS)
        cbuf[pl.ds(g * L, L)] = xv + lo - hi
        local = rows - r0
        m = (local >= 0) & (local < n_owned)
        cols = g * L + lane
        lc = jnp.where(m, local, STAGE - 1)
        plsc.store_scatter(rowcol, [lc], cols, mask=m)
        segidx = lc * L + (cols & (L - 1))
        plsc.store_scatter(segs, [segidx], err, mask=m)
        return carry + incl[L - 1]

    lax.fori_loop(0, NG, col_group, jnp.int32(0))

    seg_copies = []
    for k in range(N_CHUNKS):
        big[k].wait()
        kg = k * CHUNK // L
        cv = rowcol[pl.ds(kg * L, L)]
        for j in range(CHUNK):
            row = k * CHUNK + j
            col = cv[(k * CHUNK + j) % L]
            ws = jnp.maximum(col, 0) // L * L
            seg_copies.append(
                pltpu.async_copy(
                    segs.at[pl.ds(row * L, L)],
                    out_hbm.at[r0 + row, pl.ds(ws, L)], sems))
    for cp in seg_copies:
        cp.wait()

    @pl.when(wid == NW - 1)
    def _():
        pltpu.async_copy(zbuf.at[pl.ds(0, 1)],
                         out_hbm.at[pl.ds(N, 1)], semb.at[0]).wait()
        cv = rowcol[pl.ds(ROWS_PER_W, L)]
        col = cv[0]
        ws = jnp.maximum(col, 0) // L * L
        pltpu.async_copy(segs.at[pl.ds(ROWS_PER_W * L, L)],
                         out_hbm.at[N, pl.ds(ws, L)], sems).wait()

    @pl.when(wid == 0)
    def _():
        pltpu.async_copy(cbuf, out_hbm.at[0], semb.at[1]).wait()


@jax.jit
def kernel(x):
    C, H, W = x.shape
    xf = x.reshape(N)
    mesh = plsc.VectorSubcoreMesh(core_axis_name="c", subcore_axis_name="s")
    z = pl.kernel(
        _sc_body,
        mesh=mesh,
        compiler_params=pltpu.CompilerParams(
            needs_layout_passes=False,
            disable_bounds_checks=True,
            disable_semaphore_checks=True,
        ),
        out_type=jax.ShapeDtypeStruct((N_ROWS, N), jnp.float32),
        scratch_types=[
            pltpu.VMEM((N,), jnp.float32),
            pltpu.VMEM((N,), jnp.float32),
            pltpu.VMEM((STAGE,), jnp.int32),
            pltpu.VMEM((STAGE * L,), jnp.float32),
            pltpu.VMEM((CHUNK, N), jnp.float32),
            pltpu.SemaphoreType.DMA,
            pltpu.SemaphoreType.DMA((N_CHUNKS,)),
            pltpu.SemaphoreType.DMA,
        ],
    )(xf)
    return z.reshape(N_ROWS, C, H, W)


# speedup vs baseline: 1.0054x; 1.0054x over previous
import jax
import jax.numpy as jnp
from jax import lax
from jax.experimental import pallas as pl
from jax.experimental.pallas import tpu as pltpu
from jax.experimental.pallas import tpu_sc as plsc

EPS = 0.1
N = 3072
N_ROWS = N + 1
NW = 32
ROWS_PER_W = N // NW
CHUNK = 8
N_CHUNKS = ROWS_PER_W // CHUNK
L = 16
NG = N // L
STAGE = ROWS_PER_W + L


def _prep_kernel(x_ref, center_ref, err_ref, rows_ref):
    xv = x_ref[...]
    lo = jnp.maximum(EPS - xv, 0.0) * 0.5
    hi = jnp.maximum(xv - (1.0 - EPS), 0.0) * 0.5
    err = EPS - lo - hi
    cond = err >= 0.0
    ci = cond.astype(jnp.int32)
    lane = jax.lax.broadcasted_iota(jnp.int32, (1, N), 1)
    incl = ci
    s = 1
    while s < N:
        shifted = pltpu.roll(incl, s, axis=1)
        incl = incl + jnp.where(lane >= s, shifted, 0)
        s *= 2
    center_ref[...] = xv + lo - hi
    err_ref[...] = err
    rows_ref[...] = jnp.where(cond, incl, N_ROWS)


def _sc_body(x_hbm, out_hbm,
             xbuf, cbuf, rowcol, segs, zbuf,
             semi, semb, sems):
    wid = lax.axis_index("s") * 2 + lax.axis_index("c")
    r0 = wid * ROWS_PER_W
    n_owned = ROWS_PER_W + (wid + 1) // NW

    cp_x = pltpu.async_copy(x_hbm, xbuf, semi)

    lane = lax.iota(jnp.int32, L)
    z16 = jnp.zeros((L,), jnp.float32)
    neg16 = jnp.full((L,), -1, jnp.int32)

    def zero_row(i, _):
        def zero_cols(g, __):
            base = g * 4 * L
            zbuf[i, pl.ds(base, L)] = z16
            zbuf[i, pl.ds(base + L, L)] = z16
            zbuf[i, pl.ds(base + 2 * L, L)] = z16
            zbuf[i, pl.ds(base + 3 * L, L)] = z16
            return __

        return lax.fori_loop(0, NG // 4, zero_cols, _)

    lax.fori_loop(0, CHUNK, zero_row, 0)

    big = [
        pltpu.async_copy(zbuf, out_hbm.at[pl.ds(r0 + k * CHUNK, CHUNK)], semb)
        for k in range(N_CHUNKS)
    ]

    def zero_stage(j, _):
        base = j * 4 * L
        segs[pl.ds(base, L)] = z16
        segs[pl.ds(base + L, L)] = z16
        segs[pl.ds(base + 2 * L, L)] = z16
        segs[pl.ds(base + 3 * L, L)] = z16
        return _

    lax.fori_loop(0, STAGE // 4, zero_stage, 0)

    def zero_rc(j, _):
        rowcol[pl.ds(j * L, L)] = neg16
        return _

    lax.fori_loop(0, STAGE // L, zero_rc, 0)

    cp_x.wait()

    def col_group(g, carry):
        xv = xbuf[pl.ds(g * L, L)]
        lo = jnp.maximum(EPS - xv, 0.0) * 0.5
        hi = jnp.maximum(xv - (1.0 - EPS), 0.0) * 0.5
        err = EPS - lo - hi
        cond = err >= 0.0
        ci = jnp.where(cond, 1, 0)
        incl = plsc.cumsum(ci)
        rows = jnp.where(cond, carry + incl, N_ROWS)
        cbuf[pl.ds(g * L, L)] = xv + lo - hi
        local = rows - r0
        m = (local >= 0) & (local < n_owned)
        cols = g * L + lane
        lc = jnp.where(m, local, STAGE - 1)
        plsc.store_scatter(rowcol, [lc], cols, mask=m)
        segidx = lc * L + (cols & (L - 1))
        plsc.store_scatter(segs, [segidx], err, mask=m)
        return carry + incl[L - 1]

    lax.fori_loop(0, NG, col_group, jnp.int32(0))

    for cp in big:
        cp.wait()
    seg_copies = []
    for k in range(ROWS_PER_W // L):
        cv = rowcol[pl.ds(k * L, L)]
        for j in range(L):
            row = k * L + j
            col = cv[j]
            ws = jnp.maximum(col, 0) // L * L
            seg_copies.append(
                pltpu.async_copy(
                    segs.at[pl.ds(row * L, L)],
                    out_hbm.at[r0 + row, pl.ds(ws, L)], sems))
    for cp in seg_copies:
        cp.wait()

    @pl.when(wid == NW - 1)
    def _():
        pltpu.async_copy(zbuf.at[pl.ds(0, 1)],
                         out_hbm.at[pl.ds(N, 1)], semb).wait()
        cv = rowcol[pl.ds(ROWS_PER_W, L)]
        col = cv[0]
        ws = jnp.maximum(col, 0) // L * L
        pltpu.async_copy(segs.at[pl.ds(ROWS_PER_W * L, L)],
                         out_hbm.at[N, pl.ds(ws, L)], sems).wait()

    @pl.when(wid == 0)
    def _():
        pltpu.async_copy(cbuf, out_hbm.at[0], semb).wait()


@jax.jit
def kernel(x):
    C, H, W = x.shape
    xf = x.reshape(N)
    mesh = plsc.VectorSubcoreMesh(core_axis_name="c", subcore_axis_name="s")
    z = pl.kernel(
        _sc_body,
        mesh=mesh,
        compiler_params=pltpu.CompilerParams(
            needs_layout_passes=False,
            disable_bounds_checks=True,
            disable_semaphore_checks=True,
        ),
        out_type=jax.ShapeDtypeStruct((N_ROWS, N), jnp.float32),
        scratch_types=[
            pltpu.VMEM((N,), jnp.float32),
            pltpu.VMEM((N,), jnp.float32),
            pltpu.VMEM((STAGE,), jnp.int32),
            pltpu.VMEM((STAGE * L,), jnp.float32),
            pltpu.VMEM((CHUNK, N), jnp.float32),
            pltpu.SemaphoreType.DMA,
            pltpu.SemaphoreType.DMA,
            pltpu.SemaphoreType.DMA,
        ],
    )(xf)
    return z.reshape(N_ROWS, C, H, W)
